# baseline (speedup 1.0000x reference)
import functools

import jax
import jax.numpy as jnp
from jax import lax
from jax.experimental import pallas as pl
from jax.experimental.pallas import tpu as pltpu
from jax.experimental.pallas import tpu_sc as plsc


def _make_sc_gather(V, R, N):
    info = plsc.get_sparse_core_info()
    NC, NS = info.num_cores, info.num_subcores
    NW = NC * NS
    assert N % NW == 0
    per_w = N // NW
    chunk = 3200
    assert per_w % chunk == 0
    n_chunks = per_w // chunk
    mesh = plsc.VectorSubcoreMesh(core_axis_name="c", subcore_axis_name="s")

    @functools.partial(
        pl.kernel,
        mesh=mesh,
        compiler_params=pltpu.CompilerParams(use_tc_tiling_on_sc=False),
        out_type=jax.ShapeDtypeStruct((N, R), jnp.float32),
        scratch_types=[
            pltpu.VMEM((chunk,), jnp.int32),
            pltpu.VMEM((chunk, R), jnp.float32),
            pltpu.SemaphoreType.DMA,
        ],
    )
    def k(table_hbm, idx_hbm, out_hbm, idx_v, rows_v, sem):
        wid = lax.axis_index("s") * NC + lax.axis_index("c")
        base = wid * per_w

        def body(c, _):
            off = base + c * chunk
            pltpu.sync_copy(idx_hbm.at[pl.ds(off, chunk)], idx_v)
            pltpu.async_copy(table_hbm.at[idx_v], rows_v, sem).wait()
            pltpu.sync_copy(rows_v, out_hbm.at[pl.ds(off, chunk)])
            return ()

        lax.fori_loop(0, n_chunks, body, ())

    return k


def _mm_body(x_ref, w_ref, g_ref, o_ref):
    z = jax.nn.sigmoid(g_ref[...])
    w = w_ref[...] * z.reshape(-1, 1)
    o_ref[...] = jnp.dot(x_ref[...], w, preferred_element_type=jnp.float32)


def _gated_matmul(mid, right, gate_logits, block_rows=2048):
    N, R = mid.shape
    H = right.shape[1]
    grid = N // block_rows
    return pl.pallas_call(
        _mm_body,
        grid=(grid,),
        in_specs=[
            pl.BlockSpec((block_rows, R), lambda i: (i, 0)),
            pl.BlockSpec((R, H), lambda i: (0, 0)),
            pl.BlockSpec((1, R), lambda i: (0, 0)),
        ],
        out_specs=pl.BlockSpec((block_rows, H), lambda i: (i, 0)),
        out_shape=jax.ShapeDtypeStruct((N, H), jnp.float32),
    )(mid, right, gate_logits.reshape(1, R))


def kernel(input_ids, left, right, gate_logits):
    B, L = input_ids.shape
    V, R = left.shape
    H = right.shape[1]
    N = B * L
    ids_flat = input_ids.reshape(N)
    mid = _make_sc_gather(V, R, N)(left, ids_flat)
    out = _gated_matmul(mid, right, gate_logits)
    return out.reshape(B, L, H)


# speedup vs baseline: 1.4402x; 1.4402x over previous
import functools

import jax
import jax.numpy as jnp
from jax import lax
from jax.experimental import pallas as pl
from jax.experimental.pallas import tpu as pltpu
from jax.experimental.pallas import tpu_sc as plsc


def _make_sc_gather(V, R, N):
    info = plsc.get_sparse_core_info()
    NC, NS = info.num_cores, info.num_subcores
    NW = NC * NS
    assert N % NW == 0
    per_w = N // NW
    chunk = 3200
    assert per_w % chunk == 0
    n_chunks = per_w // chunk
    mesh = plsc.VectorSubcoreMesh(core_axis_name="c", subcore_axis_name="s")

    @functools.partial(
        pl.kernel,
        mesh=mesh,
        compiler_params=pltpu.CompilerParams(use_tc_tiling_on_sc=False),
        out_type=jax.ShapeDtypeStruct((N, R), jnp.float32),
        scratch_types=[
            pltpu.VMEM((chunk,), jnp.int32),
            pltpu.VMEM((chunk, R), jnp.float32),
            pltpu.SemaphoreType.DMA,
        ],
    )
    def k(table_hbm, idx_hbm, out_hbm, idx_v, rows_v, sem):
        wid = lax.axis_index("s") * NC + lax.axis_index("c")
        base = wid * per_w

        def body(c, _):
            off = base + c * chunk
            pltpu.sync_copy(idx_hbm.at[pl.ds(off, chunk)], idx_v)
            pltpu.async_copy(table_hbm.at[idx_v], rows_v, sem).wait()
            pltpu.sync_copy(rows_v, out_hbm.at[pl.ds(off, chunk)])
            return ()

        lax.fori_loop(0, n_chunks, body, ())

    return k


def _mm_body(x_ref, w_ref, g_ref, o_ref):
    z = jax.nn.sigmoid(g_ref[...])
    w = w_ref[...] * z.reshape(-1, 1)
    x = x_ref[...]
    R = w.shape[0]
    xs = jnp.concatenate([x[:, g * R:(g + 1) * R] for g in range(4)], axis=0)
    y = jnp.dot(xs, w, preferred_element_type=jnp.float32)
    o_ref[...] = y.reshape(4, x.shape[0], w.shape[1])


def _gated_matmul(mid_packed, right, gate_logits, block_rows=512):
    NP, RP = mid_packed.shape
    R, H = right.shape
    grid = NP // block_rows
    out = pl.pallas_call(
        _mm_body,
        grid=(grid,),
        in_specs=[
            pl.BlockSpec((block_rows, RP), lambda i: (i, 0)),
            pl.BlockSpec((R, H), lambda i: (0, 0)),
            pl.BlockSpec((1, R), lambda i: (0, 0)),
        ],
        out_specs=pl.BlockSpec((4, block_rows, H), lambda i: (0, i, 0)),
        out_shape=jax.ShapeDtypeStruct((4, NP, H), jnp.float32),
    )(mid_packed, right, gate_logits.reshape(1, R))
    return out.reshape(4 * NP, H)


def kernel(input_ids, left, right, gate_logits):
    B, L = input_ids.shape
    V, R = left.shape
    H = right.shape[1]
    N = B * L
    ids_t = input_ids.T.reshape(N)
    ids_perm = ids_t.reshape(4, N // 4).T.reshape(N)
    mid = _make_sc_gather(V, R, N)(left, ids_perm)
    mid_packed = mid.reshape(N // 4, 4 * R)
    y = _gated_matmul(mid_packed, right, gate_logits)
    return y.reshape(L, B, H).transpose(1, 0, 2)
